# baseline (speedup 1.0000x reference)
import functools

import jax
import jax.numpy as jnp
from jax import lax
from jax.experimental import pallas as pl
from jax.experimental.pallas import tpu as pltpu
from jax.experimental.pallas import tpu_sc as plsc

N = 10000
D = 128
E = 320000

NW = 32
C = 64
NCH = 162
EPT = NCH * C
E_PAD = NW * EPT
NPAD = 10240
SLOP = 10008
RPT = NPAD // 16
NTRI = NCH // 3


def _sc_aggregate(h, p_src, p_dst, src, dst):
  mesh = plsc.VectorSubcoreMesh(core_axis_name="c", subcore_axis_name="s")

  @functools.partial(
      pl.kernel,
      out_type=[
          jax.ShapeDtypeStruct((2, NPAD, D), jnp.float32),
          jax.ShapeDtypeStruct((2, NPAD), jnp.float32),
      ],
      mesh=mesh,
      compiler_params=pltpu.CompilerParams(needs_layout_passes=False),
      scratch_types=[
          pltpu.VMEM((NPAD,), jnp.float32),
          pltpu.VMEM((NPAD,), jnp.float32),
          [pltpu.VMEM((C, D), jnp.float32)] * 3,
          [pltpu.VMEM((C,), jnp.float32)] * 3,
          [pltpu.VMEM((C,), jnp.int32)] * 3,
          [pltpu.VMEM((C,), jnp.int32)] * 3,
          [pltpu.VMEM((C,), jnp.int32)] * 3,
          pltpu.VMEM((RPT,), jnp.float32),
          pltpu.VMEM_SHARED((NPAD, D), jnp.float32),
          pltpu.VMEM_SHARED((NPAD,), jnp.float32),
          [pltpu.SemaphoreType.DMA] * 3,
          [pltpu.SemaphoreType.DMA] * 3,
          [pltpu.SemaphoreType.DMA] * 3,
      ],
  )
  def k(h_hbm, ps_hbm, pd_hbm, src_hbm, dst_hbm, acc_hbm, den_hbm,
        psrc_t, pdst_t, rows, exb, srcc, dstc, dsts, zscal,
        acc_sh, den_sh, semI, semG, semS):
    c = lax.axis_index("c")
    s = lax.axis_index("s")
    wid = s * 2 + c
    ebase = wid * EPT
    zv = jnp.zeros((16,), jnp.float32)

    def issue_idx(j, p):
      pltpu.async_copy(src_hbm.at[pl.ds(ebase + j * C, C)], srcc[p], semI[p])
      pltpu.async_copy(dst_hbm.at[pl.ds(ebase + j * C, C)], dstc[p], semI[p])

    def wait_idx(j, p):
      pltpu.make_async_copy(
          src_hbm.at[pl.ds(ebase + j * C, C)], srcc[p], semI[p]).wait()
      pltpu.make_async_copy(
          dst_hbm.at[pl.ds(ebase + j * C, C)], dstc[p], semI[p]).wait()

    def compute_ex(p):
      for g in range(C // 16):
        si = srcc[p][pl.ds(g * 16, 16)]
        di = dstc[p][pl.ds(g * 16, 16)]
        dsts[p][pl.ds(g * 16, 16)] = di
        a = plsc.load_gather(psrc_t, [si]) + plsc.load_gather(pdst_t, [di])
        a = jnp.where(a > 0, a, 0.2 * a)
        exb[p][pl.ds(g * 16, 16)] = jnp.exp(a)

    def issue_gather(p):
      pltpu.async_copy(h_hbm.at[srcc[p]], rows[p], semG[p])

    def wait_gather(p):
      pltpu.make_async_copy(h_hbm.at[srcc[p]], rows[p], semG[p]).wait()

    def scale_rows(p):
      def scale(g, carry2):
        for e in range(16):
          r = g * 16 + e
          bc = plsc.load_gather(exb[p], [jnp.full((16,), r, jnp.int32)])
          for g2 in range(D // 16):
            rows[p][r, pl.ds(g2 * 16, 16)] = (
                rows[p][r, pl.ds(g2 * 16, 16)] * bc)
        return carry2

      lax.fori_loop(0, C // 16, scale, 0)

    def issue_scatter(p):
      pltpu.async_copy(rows[p], acc_sh.at[dsts[p]], semS[p], add=True)
      pltpu.async_copy(exb[p], den_sh.at[dsts[p]], semS[p], add=True)

    def wait_scatter(p):
      pltpu.make_async_copy(rows[p], acc_sh.at[dsts[p]], semS[p]).wait()
      pltpu.make_async_copy(exb[p], den_sh.at[dsts[p]], semS[p]).wait()

    pltpu.sync_copy(ps_hbm, psrc_t)
    pltpu.sync_copy(pd_hbm, pdst_t)

    def zrow(p):
      def zr(i, carry):
        for g in range(D // 16):
          rows[p][i, pl.ds(g * 16, 16)] = zv
        return carry
      lax.fori_loop(0, C, zr, 0)

    for p in range(3):
      zrow(p)
    for g in range(C // 16):
      exb[1][pl.ds(g * 16, 16)] = zv
      exb[2][pl.ds(g * 16, 16)] = zv
      ii = lax.iota(jnp.int32, 16) + g * 16
      dsts[1][pl.ds(g * 16, 16)] = ii
      dsts[2][pl.ds(g * 16, 16)] = ii

    def zs(i, carry):
      zscal[pl.ds(i * 16, 16)] = zv
      return carry

    lax.fori_loop(0, RPT // 16, zs, 0)

    base_row = s * RPT
    for kk in range(RPT // C):
      pltpu.sync_copy(rows[0], acc_sh.at[pl.ds(base_row + kk * C, C)])
    pltpu.sync_copy(zscal, den_sh.at[pl.ds(base_row, RPT)])
    plsc.subcore_barrier()

    issue_scatter(1)
    issue_scatter(2)
    pltpu.sync_copy(src_hbm.at[pl.ds(ebase, C)], srcc[0])
    pltpu.sync_copy(dst_hbm.at[pl.ds(ebase, C)], dstc[0])
    compute_ex(0)
    issue_gather(0)
    issue_idx(1, 1)

    def tri(t, carry):
      for p in range(3):
        q = (p + 1) % 3
        r = (p + 2) % 3
        j = 3 * t + p
        jn = j + 1
        jn2 = j + 2

        wait_scatter(q)

        @pl.when(jn < NCH)
        def _():
          wait_idx(jn, q)
          compute_ex(q)
          issue_gather(q)

        wait_gather(p)

        @pl.when(jn2 < NCH)
        def _():
          issue_idx(jn2, r)

        scale_rows(p)
        issue_scatter(p)
      return carry

    lax.fori_loop(0, NTRI, tri, 0)

    wait_scatter(1)
    wait_scatter(2)
    plsc.subcore_barrier()

    @pl.when(s == 0)
    def _():
      pltpu.sync_copy(acc_sh, acc_hbm.at[c])
      pltpu.sync_copy(den_sh, den_hbm.at[c])

  return k(h, p_src, p_dst, src, dst)


def _tc_entry(x, W, A):
  def body(x_ref, w_ref, a_ref, h_ref, p_ref):
    h = jnp.dot(x_ref[...], w_ref[...], preferred_element_type=jnp.float32)
    h_ref[...] = h
    p_ref[...] = jnp.dot(h, a_ref[...], preferred_element_type=jnp.float32)

  return pl.pallas_call(
      body,
      out_shape=[
          jax.ShapeDtypeStruct((NPAD, D), jnp.float32),
          jax.ShapeDtypeStruct((NPAD, D), jnp.float32),
      ],
  )(x, W, A)


def _tc_mid(acc, den, b, W, A):
  def body(acc_ref, den_ref, b_ref, w_ref, a_ref, h_ref, p_ref):
    agg = acc_ref[0] + acc_ref[1]
    dsum = den_ref[0] + den_ref[1]
    hin = agg / (dsum[:, None] + 1e-16) + b_ref[...]
    hin = jnp.maximum(hin, 0.0)
    h2 = jnp.dot(hin, w_ref[...], preferred_element_type=jnp.float32)
    h_ref[...] = h2
    p_ref[...] = jnp.dot(h2, a_ref[...], preferred_element_type=jnp.float32)

  return pl.pallas_call(
      body,
      out_shape=[
          jax.ShapeDtypeStruct((NPAD, D), jnp.float32),
          jax.ShapeDtypeStruct((NPAD, D), jnp.float32),
      ],
  )(acc, den, b, W, A)


def _tc_out(acc, den, b):
  def body(acc_ref, den_ref, b_ref, o_ref):
    agg = acc_ref[0] + acc_ref[1]
    dsum = den_ref[0] + den_ref[1]
    o = agg / (dsum[:, None] + 1e-16) + b_ref[...]
    m = jnp.max(o, axis=-1, keepdims=True)
    ex = jnp.exp(o - m)
    o_ref[...] = (o - m) - jnp.log(jnp.sum(ex, axis=-1, keepdims=True))

  return pl.pallas_call(
      body,
      out_shape=jax.ShapeDtypeStruct((NPAD, D), jnp.float32),
  )(acc, den, b)


def kernel(x, edge_index, W1, a1_src, a1_dst, b1, W2, a2_src, a2_dst, b2):
  xp = jnp.pad(x.astype(jnp.float32), ((0, NPAD - N), (0, 0)))
  src = jnp.pad(edge_index[0].astype(jnp.int32), (0, E_PAD - E),
                constant_values=0)
  dst = jnp.pad(edge_index[1].astype(jnp.int32), (0, E_PAD - E),
                constant_values=SLOP)

  A1 = jnp.zeros((D, D), jnp.float32).at[:, 0].set(a1_src).at[:, 1].set(a1_dst)
  A2 = jnp.zeros((D, D), jnp.float32).at[:, 0].set(a2_src).at[:, 1].set(a2_dst)
  b1r = b1.reshape(1, D)
  b2r = b2.reshape(1, D)

  h1, P1 = _tc_entry(xp, W1, A1)
  acc1, den1 = _sc_aggregate(h1, P1[:, 0], P1[:, 1], src, dst)
  h2, P2 = _tc_mid(acc1, den1, b1r, W2, A2)
  acc2, den2 = _sc_aggregate(h2, P2[:, 0], P2[:, 1], src, dst)
  out = _tc_out(acc2, den2, b2r)
  return out[:N]


# speedup vs baseline: 1.4893x; 1.4893x over previous
import functools

import jax
import jax.numpy as jnp
from jax import lax
from jax.experimental import pallas as pl
from jax.experimental.pallas import tpu as pltpu
from jax.experimental.pallas import tpu_sc as plsc

N = 10000
D = 128
E = 320000

NW = 32
C = 96
NCH = 106
EPT = NCH * C
E_PAD = NW * EPT
NA = 10016
ND = 10240
SLOP = 10008
ART = NA // 16
DRT = ND // 16


def _sc_aggregate(h, p_src, p_dst, inter):
  mesh = plsc.VectorSubcoreMesh(core_axis_name="c", subcore_axis_name="s")

  @functools.partial(
      pl.kernel,
      out_type=[
          jax.ShapeDtypeStruct((2, NA, D), jnp.float32),
          jax.ShapeDtypeStruct((2, ND), jnp.float32),
      ],
      mesh=mesh,
      compiler_params=pltpu.CompilerParams(needs_layout_passes=False),
      scratch_types=[
          pltpu.VMEM((NA,), jnp.float32),
          pltpu.VMEM((NA,), jnp.float32),
          pltpu.VMEM((2 * C,), jnp.int32),
          [pltpu.VMEM((C, D), jnp.float32)] * 2,
          [pltpu.VMEM((C,), jnp.float32)] * 2,
          [pltpu.VMEM((C,), jnp.int32)] * 2,
          [pltpu.VMEM((C,), jnp.int32)] * 2,
          pltpu.VMEM((DRT,), jnp.float32),
          pltpu.VMEM_SHARED((NA, D), jnp.float32),
          pltpu.VMEM_SHARED((ND,), jnp.float32),
          [pltpu.SemaphoreType.DMA] * 2,
      ],
  )
  def k(h_hbm, ps_hbm, pd_hbm, inter_hbm, acc_hbm, den_hbm,
        psrc_t, pdst_t, idxb, rows, exb, srcc, dsts, zscal,
        acc_sh, den_sh, semG):
    c = lax.axis_index("c")
    s = lax.axis_index("s")
    wid = s * 2 + c
    kbase = wid * NCH
    zv = jnp.zeros((16,), jnp.float32)

    def fetch_prep(j, p):
      pltpu.sync_copy(inter_hbm.at[pl.ds((kbase + j) * 2 * C, 2 * C)], idxb)
      for g in range(C // 16):
        si = idxb[pl.ds(g * 16, 16)]
        di = idxb[pl.ds(C + g * 16, 16)]
        srcc[p][pl.ds(g * 16, 16)] = si
        dsts[p][pl.ds(g * 16, 16)] = di
        a = plsc.load_gather(psrc_t, [si]) + plsc.load_gather(pdst_t, [di])
        a = jnp.where(a > 0, a, 0.2 * a)
        exb[p][pl.ds(g * 16, 16)] = jnp.exp(a)

    def issue_gather(p):
      pltpu.async_copy(h_hbm.at[srcc[p]], rows[p], semG[p])

    def wait_gather(p):
      pltpu.make_async_copy(h_hbm.at[srcc[p]], rows[p], semG[p]).wait()

    def scale_rows(p):
      def scale(g, carry):
        for e in range(16):
          r = g * 16 + e
          bc = plsc.load_gather(exb[p], [jnp.full((16,), r, jnp.int32)])
          for g2 in range(D // 16):
            rows[p][r, pl.ds(g2 * 16, 16)] = (
                rows[p][r, pl.ds(g2 * 16, 16)] * bc)
        return carry

      lax.fori_loop(0, C // 16, scale, 0)

    def scatter(p):
      pltpu.sync_copy(rows[p], acc_sh.at[dsts[p]], add=True)
      pltpu.sync_copy(exb[p], den_sh.at[dsts[p]], add=True)

    pltpu.sync_copy(ps_hbm, psrc_t)
    pltpu.sync_copy(pd_hbm, pdst_t)

    def zrow(i, carry):
      for g in range(D // 16):
        rows[0][i, pl.ds(g * 16, 16)] = zv
      return carry

    lax.fori_loop(0, C, zrow, 0)

    def zs(i, carry):
      zscal[pl.ds(i * 16, 16)] = zv
      return carry

    lax.fori_loop(0, DRT // 16, zs, 0)

    arow = s * ART
    for kk in range(ART // C):
      pltpu.sync_copy(rows[0], acc_sh.at[pl.ds(arow + kk * C, C)])
    pltpu.sync_copy(rows[0].at[pl.ds(0, ART % C)],
                    acc_sh.at[pl.ds(arow + (ART // C) * C, ART % C)])
    pltpu.sync_copy(zscal, den_sh.at[pl.ds(s * DRT, DRT)])
    plsc.subcore_barrier()

    fetch_prep(0, 0)
    issue_gather(0)

    def pair(t, carry):
      fetch_prep(2 * t + 1, 1)
      wait_gather(0)
      issue_gather(1)
      scale_rows(0)
      scatter(0)
      fetch_prep(jnp.minimum(2 * t + 2, NCH - 2), 0)
      wait_gather(1)
      issue_gather(0)
      scale_rows(1)
      scatter(1)
      return carry

    lax.fori_loop(0, NCH // 2, pair, 0)
    wait_gather(0)
    plsc.subcore_barrier()

    @pl.when(s == 0)
    def _():
      pltpu.sync_copy(acc_sh, acc_hbm.at[c])
      pltpu.sync_copy(den_sh, den_hbm.at[c])

  return k(h, p_src, p_dst, inter)


def _tc_entry(x, W, A):
  def body(x_ref, w_ref, a_ref, h_ref, p_ref):
    h = jnp.dot(x_ref[...], w_ref[...], preferred_element_type=jnp.float32)
    h_ref[...] = h
    p_ref[...] = jnp.dot(h, a_ref[...], preferred_element_type=jnp.float32)

  return pl.pallas_call(
      body,
      out_shape=[
          jax.ShapeDtypeStruct((NA, D), jnp.float32),
          jax.ShapeDtypeStruct((NA, D), jnp.float32),
      ],
  )(x, W, A)


def _tc_mid(acc, den, b, W, A):
  def body(acc_ref, den_ref, b_ref, w_ref, a_ref, h_ref, p_ref):
    agg = acc_ref[0] + acc_ref[1]
    dsum = (den_ref[0] + den_ref[1])[:NA]
    hin = agg / (dsum[:, None] + 1e-16) + b_ref[...]
    hin = jnp.maximum(hin, 0.0)
    h2 = jnp.dot(hin, w_ref[...], preferred_element_type=jnp.float32)
    h_ref[...] = h2
    p_ref[...] = jnp.dot(h2, a_ref[...], preferred_element_type=jnp.float32)

  return pl.pallas_call(
      body,
      out_shape=[
          jax.ShapeDtypeStruct((NA, D), jnp.float32),
          jax.ShapeDtypeStruct((NA, D), jnp.float32),
      ],
  )(acc, den, b, W, A)


def _tc_out(acc, den, b):
  def body(acc_ref, den_ref, b_ref, o_ref):
    agg = acc_ref[0] + acc_ref[1]
    dsum = (den_ref[0] + den_ref[1])[:NA]
    o = agg / (dsum[:, None] + 1e-16) + b_ref[...]
    m = jnp.max(o, axis=-1, keepdims=True)
    ex = jnp.exp(o - m)
    o_ref[...] = (o - m) - jnp.log(jnp.sum(ex, axis=-1, keepdims=True))

  return pl.pallas_call(
      body,
      out_shape=jax.ShapeDtypeStruct((NA, D), jnp.float32),
  )(acc, den, b)


def kernel(x, edge_index, W1, a1_src, a1_dst, b1, W2, a2_src, a2_dst, b2):
  xp = jnp.pad(x.astype(jnp.float32), ((0, NA - N), (0, 0)))
  src = jnp.pad(edge_index[0].astype(jnp.int32), (0, E_PAD - E),
                constant_values=0)
  dst = jnp.pad(edge_index[1].astype(jnp.int32), (0, E_PAD - E),
                constant_values=SLOP)
  inter = jnp.concatenate(
      [src.reshape(-1, C), dst.reshape(-1, C)], axis=1).reshape(-1)

  A1 = jnp.zeros((D, D), jnp.float32).at[:, 0].set(a1_src).at[:, 1].set(a1_dst)
  A2 = jnp.zeros((D, D), jnp.float32).at[:, 0].set(a2_src).at[:, 1].set(a2_dst)
  b1r = b1.reshape(1, D)
  b2r = b2.reshape(1, D)

  h1, P1 = _tc_entry(xp, W1, A1)
  acc1, den1 = _sc_aggregate(h1, P1[:, 0], P1[:, 1], inter)
  h2, P2 = _tc_mid(acc1, den1, b1r, W2, A2)
  acc2, den2 = _sc_aggregate(h2, P2[:, 0], P2[:, 1], inter)
  out = _tc_out(acc2, den2, b2r)
  return out[:N]


# speedup vs baseline: 1.6602x; 1.1147x over previous
import functools

import jax
import jax.numpy as jnp
from jax import lax
from jax.experimental import pallas as pl
from jax.experimental.pallas import tpu as pltpu
from jax.experimental.pallas import tpu_sc as plsc

N = 10000
D = 128
E = 320000

NW = 32
C = 96
NCH = 106
EPT = NCH * C
E_PAD = NW * EPT
NA = 10016
ND = 10240
SLOP = 10008
ART = NA // 16
DRT = ND // 16


def _sc_aggregate(h, p_src, p_dst, inter):
  mesh = plsc.VectorSubcoreMesh(core_axis_name="c", subcore_axis_name="s")

  @functools.partial(
      pl.kernel,
      out_type=[
          jax.ShapeDtypeStruct((2, NA, D), jnp.float32),
          jax.ShapeDtypeStruct((2, ND), jnp.float32),
      ],
      mesh=mesh,
      compiler_params=pltpu.CompilerParams(
          needs_layout_passes=False, use_tc_tiling_on_sc=False),
      scratch_types=[
          pltpu.VMEM((NA,), jnp.float32),
          pltpu.VMEM((NA,), jnp.float32),
          pltpu.VMEM((2 * C,), jnp.int32),
          [pltpu.VMEM((C, D // 2), jnp.int32)] * 2,
          pltpu.VMEM((C, D), jnp.float32),
          [pltpu.VMEM((C,), jnp.float32)] * 2,
          [pltpu.VMEM((C,), jnp.int32)] * 2,
          [pltpu.VMEM((C,), jnp.int32)] * 2,
          pltpu.VMEM((DRT,), jnp.float32),
          pltpu.VMEM_SHARED((NA, D), jnp.float32),
          pltpu.VMEM_SHARED((ND,), jnp.float32),
          [pltpu.SemaphoreType.DMA] * 2,
      ],
  )
  def k(h_hbm, ps_hbm, pd_hbm, inter_hbm, acc_hbm, den_hbm,
        psrc_t, pdst_t, idxb, rows, rowsF, exb, srcc, dsts, zscal,
        acc_sh, den_sh, semG):
    c = lax.axis_index("c")
    s = lax.axis_index("s")
    wid = s * 2 + c
    kbase = wid * NCH
    zv = jnp.zeros((16,), jnp.float32)

    def fetch_prep(j, p):
      pltpu.sync_copy(inter_hbm.at[pl.ds((kbase + j) * 2 * C, 2 * C)], idxb)
      for g in range(C // 16):
        si = idxb[pl.ds(g * 16, 16)]
        di = idxb[pl.ds(C + g * 16, 16)]
        srcc[p][pl.ds(g * 16, 16)] = si
        dsts[p][pl.ds(g * 16, 16)] = di
        a = plsc.load_gather(psrc_t, [si]) + plsc.load_gather(pdst_t, [di])
        a = jnp.where(a > 0, a, 0.2 * a)
        exb[p][pl.ds(g * 16, 16)] = jnp.exp(a)

    def issue_gather(p):
      pltpu.async_copy(h_hbm.at[srcc[p]], rows[p], semG[p])

    def wait_gather(p):
      pltpu.make_async_copy(h_hbm.at[srcc[p]], rows[p], semG[p]).wait()

    def scale_rows(p):
      def scale(g, carry):
        for e in range(16):
          r = g * 16 + e
          bc = plsc.load_gather(exb[p], [jnp.full((16,), r, jnp.int32)])
          for g2 in range(D // 32):
            packed = plsc.bitcast(rows[p][r, pl.ds(g2 * 16, 16)],
                                  jnp.bfloat16)
            lo, hi = plsc.unpack(packed, format=plsc.PackFormat.INTERLEAVED)
            rowsF[r, pl.ds(g2 * 16, 16)] = lo * bc
            rowsF[r, pl.ds(D // 2 + g2 * 16, 16)] = hi * bc
        return carry

      lax.fori_loop(0, C // 16, scale, 0)

    def scatter(p):
      pltpu.sync_copy(rowsF, acc_sh.at[dsts[p]], add=True)
      pltpu.sync_copy(exb[p], den_sh.at[dsts[p]], add=True)

    pltpu.sync_copy(ps_hbm, psrc_t)
    pltpu.sync_copy(pd_hbm, pdst_t)

    def zrow(i, carry):
      for g in range(D // 16):
        rowsF[i, pl.ds(g * 16, 16)] = zv
      return carry

    lax.fori_loop(0, C, zrow, 0)

    def zs(i, carry):
      zscal[pl.ds(i * 16, 16)] = zv
      return carry

    lax.fori_loop(0, DRT // 16, zs, 0)

    arow = s * ART
    for kk in range(ART // C):
      pltpu.sync_copy(rowsF, acc_sh.at[pl.ds(arow + kk * C, C)])
    pltpu.sync_copy(rowsF.at[pl.ds(0, ART % C)],
                    acc_sh.at[pl.ds(arow + (ART // C) * C, ART % C)])
    pltpu.sync_copy(zscal, den_sh.at[pl.ds(s * DRT, DRT)])
    plsc.subcore_barrier()

    fetch_prep(0, 0)
    issue_gather(0)

    def pair(t, carry):
      fetch_prep(2 * t + 1, 1)
      wait_gather(0)
      issue_gather(1)
      scale_rows(0)
      scatter(0)
      fetch_prep(jnp.minimum(2 * t + 2, NCH - 2), 0)
      wait_gather(1)
      issue_gather(0)
      scale_rows(1)
      scatter(1)
      return carry

    lax.fori_loop(0, NCH // 2, pair, 0)
    wait_gather(0)
    plsc.subcore_barrier()

    @pl.when(s == 0)
    def _():
      pltpu.sync_copy(acc_sh, acc_hbm.at[c])
      pltpu.sync_copy(den_sh, den_hbm.at[c])

  return k(h, p_src, p_dst, inter)


def _tc_entry(x, W, A):
  def body(x_ref, w_ref, a_ref, h_ref, p_ref):
    h = jnp.dot(x_ref[...], w_ref[...], preferred_element_type=jnp.float32)
    h_ref[...] = h
    p_ref[...] = jnp.dot(h, a_ref[...], preferred_element_type=jnp.float32)

  return pl.pallas_call(
      body,
      out_shape=[
          jax.ShapeDtypeStruct((NA, D), jnp.float32),
          jax.ShapeDtypeStruct((NA, D), jnp.float32),
      ],
  )(x, W, A)


def _tc_mid(acc, den, b, W, A):
  def body(acc_ref, den_ref, b_ref, w_ref, a_ref, h_ref, p_ref):
    agg = acc_ref[0] + acc_ref[1]
    dsum = (den_ref[0] + den_ref[1])[:NA]
    hin = agg / (dsum[:, None] + 1e-16) + b_ref[...]
    hin = jnp.maximum(hin, 0.0)
    h2 = jnp.dot(hin, w_ref[...], preferred_element_type=jnp.float32)
    h_ref[...] = h2
    p_ref[...] = jnp.dot(h2, a_ref[...], preferred_element_type=jnp.float32)

  return pl.pallas_call(
      body,
      out_shape=[
          jax.ShapeDtypeStruct((NA, D), jnp.float32),
          jax.ShapeDtypeStruct((NA, D), jnp.float32),
      ],
  )(acc, den, b, W, A)


def _tc_out(acc, den, b):
  def body(acc_ref, den_ref, b_ref, o_ref):
    agg = acc_ref[0] + acc_ref[1]
    dsum = (den_ref[0] + den_ref[1])[:NA]
    o = agg / (dsum[:, None] + 1e-16) + b_ref[...]
    m = jnp.max(o, axis=-1, keepdims=True)
    ex = jnp.exp(o - m)
    o_ref[...] = (o - m) - jnp.log(jnp.sum(ex, axis=-1, keepdims=True))

  return pl.pallas_call(
      body,
      out_shape=jax.ShapeDtypeStruct((NA, D), jnp.float32),
  )(acc, den, b)


def kernel(x, edge_index, W1, a1_src, a1_dst, b1, W2, a2_src, a2_dst, b2):
  xp = jnp.pad(x.astype(jnp.float32), ((0, NA - N), (0, 0)))
  src = jnp.pad(edge_index[0].astype(jnp.int32), (0, E_PAD - E),
                constant_values=0)
  dst = jnp.pad(edge_index[1].astype(jnp.int32), (0, E_PAD - E),
                constant_values=SLOP)
  inter = jnp.concatenate(
      [src.reshape(-1, C), dst.reshape(-1, C)], axis=1).reshape(-1)

  def inter_cols(h):
    hb = jnp.stack([h[:, :D // 2], h[:, D // 2:]],
                   axis=-1).astype(jnp.bfloat16)
    return jax.lax.bitcast_convert_type(hb, jnp.int32)

  A1 = jnp.zeros((D, D), jnp.float32).at[:, 0].set(a1_src).at[:, 1].set(a1_dst)
  A2 = jnp.zeros((D, D), jnp.float32).at[:, 0].set(a2_src).at[:, 1].set(a2_dst)
  b1r = b1.reshape(1, D)
  b2r = b2.reshape(1, D)

  h1, P1 = _tc_entry(xp, W1, A1)
  acc1, den1 = _sc_aggregate(inter_cols(h1), P1[:, 0], P1[:, 1], inter)
  h2, P2 = _tc_mid(acc1, den1, b1r, W2, A2)
  acc2, den2 = _sc_aggregate(inter_cols(h2), P2[:, 0], P2[:, 1], inter)
  out = _tc_out(acc2, den2, b2r)
  return out[:N]


# speedup vs baseline: 1.6792x; 1.0115x over previous
import functools

import jax
import jax.numpy as jnp
from jax import lax
from jax.experimental import pallas as pl
from jax.experimental.pallas import tpu as pltpu
from jax.experimental.pallas import tpu_sc as plsc

N = 10000
D = 128
E = 320000

NW = 32
C = 112
NCH = 92
EPT = NCH * C
E_PAD = NW * EPT
NA = 10016
ND = 10240
SLOP = 10008
ART = NA // 16
DRT = ND // 16


def _sc_aggregate(h, p_src, p_dst, src, dst):
  mesh = plsc.VectorSubcoreMesh(core_axis_name="c", subcore_axis_name="s")

  @functools.partial(
      pl.kernel,
      out_type=[
          jax.ShapeDtypeStruct((2, NA, D), jnp.float32),
          jax.ShapeDtypeStruct((2, ND), jnp.float32),
      ],
      mesh=mesh,
      compiler_params=pltpu.CompilerParams(
          needs_layout_passes=False, use_tc_tiling_on_sc=False),
      scratch_types=[
          pltpu.VMEM((NA,), jnp.float32),
          pltpu.VMEM((NA,), jnp.float32),
          [pltpu.VMEM((C, D // 2), jnp.int32)] * 2,
          pltpu.VMEM((C, D), jnp.float32),
          [pltpu.VMEM((C,), jnp.float32)] * 2,
          [pltpu.VMEM((C,), jnp.int32)] * 2,
          [pltpu.VMEM((C,), jnp.int32)] * 2,
          pltpu.VMEM((DRT,), jnp.float32),
          pltpu.VMEM_SHARED((NA, D), jnp.float32),
          pltpu.VMEM_SHARED((ND,), jnp.float32),
          [pltpu.SemaphoreType.DMA] * 2,
          [pltpu.SemaphoreType.DMA] * 2,
      ],
  )
  def k(h_hbm, ps_hbm, pd_hbm, src_hbm, dst_hbm, acc_hbm, den_hbm,
        psrc_t, pdst_t, rows, rowsF, exb, srcc, dsts, zscal,
        acc_sh, den_sh, semI, semG):
    c = lax.axis_index("c")
    s = lax.axis_index("s")
    wid = s * 2 + c
    ebase = wid * EPT
    zv = jnp.zeros((16,), jnp.float32)

    def issue_idx(j, p):
      pltpu.async_copy(src_hbm.at[pl.ds(ebase + j * C, C)], srcc[p], semI[p])
      pltpu.async_copy(dst_hbm.at[pl.ds(ebase + j * C, C)], dsts[p], semI[p])

    def wait_idx(j, p):
      pltpu.make_async_copy(
          src_hbm.at[pl.ds(ebase + j * C, C)], srcc[p], semI[p]).wait()
      pltpu.make_async_copy(
          dst_hbm.at[pl.ds(ebase + j * C, C)], dsts[p], semI[p]).wait()

    def compute_ex(p):
      for g in range(C // 16):
        si = srcc[p][pl.ds(g * 16, 16)]
        di = dsts[p][pl.ds(g * 16, 16)]
        a = plsc.load_gather(psrc_t, [si]) + plsc.load_gather(pdst_t, [di])
        a = jnp.where(a > 0, a, 0.2 * a)
        exb[p][pl.ds(g * 16, 16)] = jnp.exp(a)

    def issue_gather(p):
      pltpu.async_copy(h_hbm.at[srcc[p]], rows[p], semG[p])

    def wait_gather(p):
      pltpu.make_async_copy(h_hbm.at[srcc[p]], rows[p], semG[p]).wait()

    def scale_rows(p):
      def scale(g, carry):
        for e in range(16):
          r = g * 16 + e
          bc = plsc.load_gather(exb[p], [jnp.full((16,), r, jnp.int32)])
          for g2 in range(D // 32):
            packed = plsc.bitcast(rows[p][r, pl.ds(g2 * 16, 16)],
                                  jnp.bfloat16)
            lo, hi = plsc.unpack(packed, format=plsc.PackFormat.INTERLEAVED)
            rowsF[r, pl.ds(g2 * 16, 16)] = lo * bc
            rowsF[r, pl.ds(D // 2 + g2 * 16, 16)] = hi * bc
        return carry

      lax.fori_loop(0, C // 16, scale, 0)

    def scatter(p):
      pltpu.sync_copy(rowsF, acc_sh.at[dsts[p]], add=True)
      pltpu.sync_copy(exb[p], den_sh.at[dsts[p]], add=True)

    pltpu.sync_copy(ps_hbm, psrc_t)
    pltpu.sync_copy(pd_hbm, pdst_t)

    def zrow(i, carry):
      for g in range(D // 16):
        rowsF[i, pl.ds(g * 16, 16)] = zv
      return carry

    lax.fori_loop(0, C, zrow, 0)

    def zs(i, carry):
      zscal[pl.ds(i * 16, 16)] = zv
      return carry

    lax.fori_loop(0, DRT // 16, zs, 0)

    arow = s * ART
    for kk in range(ART // C):
      pltpu.sync_copy(rowsF, acc_sh.at[pl.ds(arow + kk * C, C)])
    pltpu.sync_copy(rowsF.at[pl.ds(0, ART % C)],
                    acc_sh.at[pl.ds(arow + (ART // C) * C, ART % C)])
    pltpu.sync_copy(zscal, den_sh.at[pl.ds(s * DRT, DRT)])
    plsc.subcore_barrier()

    pltpu.sync_copy(src_hbm.at[pl.ds(ebase, C)], srcc[0])
    pltpu.sync_copy(dst_hbm.at[pl.ds(ebase, C)], dsts[0])
    compute_ex(0)
    issue_gather(0)
    issue_idx(1, 1)

    def pair(t, carry):
      jb = 2 * t + 1
      ja2 = jnp.minimum(2 * t + 2, NCH - 2)
      jb2 = jnp.minimum(2 * t + 3, NCH - 1)
      wait_idx(jb, 1)
      compute_ex(1)
      wait_gather(0)
      issue_gather(1)
      scale_rows(0)
      scatter(0)
      issue_idx(ja2, 0)
      wait_gather(1)
      wait_idx(ja2, 0)
      compute_ex(0)
      issue_gather(0)
      scale_rows(1)
      scatter(1)
      issue_idx(jb2, 1)
      return carry

    lax.fori_loop(0, NCH // 2, pair, 0)
    wait_gather(0)
    wait_idx(NCH - 1, 1)
    plsc.subcore_barrier()

    @pl.when(s == 0)
    def _():
      pltpu.sync_copy(acc_sh, acc_hbm.at[c])
      pltpu.sync_copy(den_sh, den_hbm.at[c])

  return k(h, p_src, p_dst, src, dst)


def _tc_entry(x, W, A):
  def body(x_ref, w_ref, a_ref, h_ref, p_ref):
    h = jnp.dot(x_ref[...], w_ref[...], preferred_element_type=jnp.float32)
    h_ref[...] = h
    p_ref[...] = jnp.dot(h, a_ref[...], preferred_element_type=jnp.float32)

  return pl.pallas_call(
      body,
      out_shape=[
          jax.ShapeDtypeStruct((NA, D), jnp.float32),
          jax.ShapeDtypeStruct((NA, D), jnp.float32),
      ],
  )(x, W, A)


def _tc_mid(acc, den, b, W, A):
  def body(acc_ref, den_ref, b_ref, w_ref, a_ref, h_ref, p_ref):
    agg = acc_ref[0] + acc_ref[1]
    dsum = (den_ref[0] + den_ref[1])[:NA]
    hin = agg / (dsum[:, None] + 1e-16) + b_ref[...]
    hin = jnp.maximum(hin, 0.0)
    h2 = jnp.dot(hin, w_ref[...], preferred_element_type=jnp.float32)
    h_ref[...] = h2
    p_ref[...] = jnp.dot(h2, a_ref[...], preferred_element_type=jnp.float32)

  return pl.pallas_call(
      body,
      out_shape=[
          jax.ShapeDtypeStruct((NA, D), jnp.float32),
          jax.ShapeDtypeStruct((NA, D), jnp.float32),
      ],
  )(acc, den, b, W, A)


def _tc_out(acc, den, b):
  def body(acc_ref, den_ref, b_ref, o_ref):
    agg = acc_ref[0] + acc_ref[1]
    dsum = (den_ref[0] + den_ref[1])[:NA]
    o = agg / (dsum[:, None] + 1e-16) + b_ref[...]
    m = jnp.max(o, axis=-1, keepdims=True)
    ex = jnp.exp(o - m)
    o_ref[...] = (o - m) - jnp.log(jnp.sum(ex, axis=-1, keepdims=True))

  return pl.pallas_call(
      body,
      out_shape=jax.ShapeDtypeStruct((NA, D), jnp.float32),
  )(acc, den, b)


def kernel(x, edge_index, W1, a1_src, a1_dst, b1, W2, a2_src, a2_dst, b2):
  xp = jnp.pad(x.astype(jnp.float32), ((0, NA - N), (0, 0)))
  src = jnp.pad(edge_index[0].astype(jnp.int32), (0, E_PAD - E),
                constant_values=0)
  dst = jnp.pad(edge_index[1].astype(jnp.int32), (0, E_PAD - E),
                constant_values=SLOP)

  def inter_cols(h):
    hb = jnp.stack([h[:, :D // 2], h[:, D // 2:]],
                   axis=-1).astype(jnp.bfloat16)
    return jax.lax.bitcast_convert_type(hb, jnp.int32)

  A1 = jnp.zeros((D, D), jnp.float32).at[:, 0].set(a1_src).at[:, 1].set(a1_dst)
  A2 = jnp.zeros((D, D), jnp.float32).at[:, 0].set(a2_src).at[:, 1].set(a2_dst)
  b1r = b1.reshape(1, D)
  b2r = b2.reshape(1, D)

  h1, P1 = _tc_entry(xp, W1, A1)
  acc1, den1 = _sc_aggregate(inter_cols(h1), P1[:, 0], P1[:, 1], src, dst)
  h2, P2 = _tc_mid(acc1, den1, b1r, W2, A2)
  acc2, den2 = _sc_aggregate(inter_cols(h2), P2[:, 0], P2[:, 1], src, dst)
  out = _tc_out(acc2, den2, b2r)
  return out[:N]
